# baseline (speedup 1.0000x reference)
import functools

import jax
import jax.numpy as jnp
import numpy as np
from jax import lax
from jax.experimental import pallas as pl
from jax.experimental.pallas import tpu as pltpu
from jax.experimental.pallas import tpu_sc as plsc

N = 1024
E = 32768
H = 8
D = 128
MAX_DEG = 100
MAX_SPD = 10
MAX_TIME = 365.0

NCORES = 2
NSUB = 16
NW = NCORES * NSUB
ROWS_PER_W = N // NW
EPW = E // NW
SLOTS_PER_W = ROWS_PER_W * N
CHUNK = 4096
NCHUNKS = E // CHUNK

@functools.cache
def _mesh():
  return plsc.VectorSubcoreMesh(core_axis_name="c", subcore_axis_name="s",
                                num_cores=NCORES, num_subcores=NSUB)


def _sc_prep_body(src_hbm, dst_hbm, adj_out, indeg_out, outdeg_out,
                  src_v, dst_v, sidx2, didx2, idx1_2, idx2_2, ones_v, zbuf,
                  adj_sp, indeg_sp, outdeg_sp):
  core = lax.axis_index("c")
  sid = lax.axis_index("s")
  wid = core * NSUB + sid

  def fill(i, _):
    zbuf[pl.ds(i * 16, 16)] = jnp.zeros((16,), jnp.float32)
    return 0
  lax.fori_loop(0, 2048 // 16, fill, 0)

  def fill1(i, _):
    ones_v[pl.ds(i * 16, 16)] = jnp.ones((16,), jnp.float32)
    return 0
  lax.fori_loop(0, 128 // 16, fill1, 0)

  def z_adj(k, _):
    pltpu.sync_copy(zbuf, adj_sp.at[pl.ds(sid * (N * N // NSUB) + k * 2048, 2048)])
    return 0
  lax.fori_loop(0, (N * N // NSUB) // 2048, z_adj, 0)

  @pl.when(sid == 0)
  def _():
    pltpu.sync_copy(zbuf.at[pl.ds(0, N)], indeg_sp)
    pltpu.sync_copy(zbuf.at[pl.ds(0, N)], outdeg_sp)

  pltpu.sync_copy(src_hbm.at[pl.ds(wid * EPW, EPW)], src_v)
  pltpu.sync_copy(dst_hbm.at[pl.ds(wid * EPW, EPW)], dst_v)

  for j in range(EPW // 128):
    for q in range(8):
      i = j * 8 + q
      s16 = src_v[pl.ds(i * 16, 16)]
      d16 = dst_v[pl.ds(i * 16, 16)]
      sidx2[j, pl.ds(q * 16, 16)] = s16
      didx2[j, pl.ds(q * 16, 16)] = d16
      idx1_2[j, pl.ds(q * 16, 16)] = s16 * N + d16
      idx2_2[j, pl.ds(q * 16, 16)] = d16 * N + s16

  plsc.subcore_barrier()

  for j in range(EPW // 128):
    pltpu.sync_copy(ones_v, adj_sp.at[idx1_2.at[j]], add=True)
    pltpu.sync_copy(ones_v, adj_sp.at[idx2_2.at[j]], add=True)
    pltpu.sync_copy(ones_v, indeg_sp.at[didx2.at[j]], add=True)
    pltpu.sync_copy(ones_v, outdeg_sp.at[sidx2.at[j]], add=True)

  plsc.subcore_barrier()

  seg = N * N // NSUB
  pltpu.sync_copy(adj_sp.at[pl.ds(sid * seg, seg)],
                  adj_out.at[core, pl.ds(sid * seg, seg)])

  @pl.when(sid == 0)
  def _():
    pltpu.sync_copy(indeg_sp, indeg_out.at[core])
    pltpu.sync_copy(outdeg_sp, outdeg_out.at[core])


@functools.cache
def _sc_prep():
  return pl.kernel(
    _sc_prep_body,
    out_type=(
        jax.ShapeDtypeStruct((NCORES, N * N), jnp.float32),
        jax.ShapeDtypeStruct((NCORES, N), jnp.float32),
        jax.ShapeDtypeStruct((NCORES, N), jnp.float32),
    ),
    mesh=_mesh(),
    scratch_types=[
        pltpu.VMEM((EPW,), jnp.int32),
        pltpu.VMEM((EPW,), jnp.int32),
        pltpu.VMEM((EPW // 128, 128), jnp.int32),
        pltpu.VMEM((EPW // 128, 128), jnp.int32),
        pltpu.VMEM((EPW // 128, 128), jnp.int32),
        pltpu.VMEM((EPW // 128, 128), jnp.int32),
        pltpu.VMEM((128,), jnp.float32),
        pltpu.VMEM((2048,), jnp.float32),
        pltpu.VMEM_SHARED((N * N,), jnp.float32),
        pltpu.VMEM_SHARED((N,), jnp.float32),
        pltpu.VMEM_SHARED((N,), jnp.float32),
    ],
    compiler_params=pltpu.CompilerParams(needs_layout_passes=False),
  )


def _tc_node_body(indeg_ref, outdeg_ref, ts_ref, extra_ref,
                  dwT_ref, odwT_ref, freq_ref, sinm_ref, twT_ref, tb_ref,
                  ea_ref, w1_ref, b1_ref, w2_ref, b2_ref,
                  nodeT_ref, eb_ref):
  f32 = jnp.float32
  extra = extra_ref[0, 0]

  ind = indeg_ref[...]
  ind_row = ind[0:1, :] + ind[1:2, :] + extra
  ind_row = jnp.minimum(ind_row, float(MAX_DEG))
  outd = outdeg_ref[...]
  outd_row = jnp.minimum(outd[0:1, :] + outd[1:2, :], float(MAX_DEG))

  iota_col = lax.broadcasted_iota(jnp.int32, (D, N), 0)
  ohT_in = (iota_col == ind_row.astype(jnp.int32)).astype(f32)
  ohT_out = (iota_col == outd_row.astype(jnp.int32)).astype(f32)
  centT = (jnp.dot(dwT_ref[...], ohT_in, preferred_element_type=f32)
           + jnp.dot(odwT_ref[...], ohT_out, preferred_element_type=f32))

  ts = ts_ref[...]
  tmin = jnp.min(ts)
  td = jnp.minimum((ts - tmin) / 86400.0, MAX_TIME)
  prod = freq_ref[...] * td
  peT = jnp.where(sinm_ref[...] > 0.5, jnp.sin(prod), jnp.cos(prod))
  tempT = jnp.dot(twT_ref[...], peT, preferred_element_type=f32) + tb_ref[...]

  nodeT_ref[...] = centT + tempT

  h = jnp.maximum(
      jnp.dot(ea_ref[...], w1_ref[...], preferred_element_type=f32)
      + b1_ref[...], 0.0)
  eb_ref[...] = jnp.dot(h, w2_ref[...], preferred_element_type=f32) + b2_ref[...]


def _tc_bfs_body(adjp_ref, spdsh_ref):
  f32 = jnp.float32
  a = adjp_ref[...]
  adj_b = (a[0] + a[1]) > 0.0
  r = lax.broadcasted_iota(jnp.int32, (N, N), 0)
  c = lax.broadcasted_iota(jnp.int32, (N, N), 1)
  eye = r == c
  reach = adj_b | eye
  dist = jnp.where(eye, 0.0, jnp.where(adj_b, 1.0, float(MAX_SPD + 1)))
  adj_bf = adj_b.astype(jnp.bfloat16)
  for k in range(2, MAX_SPD + 1):
    cnt = jnp.dot(reach.astype(jnp.bfloat16), adj_bf,
                  preferred_element_type=f32)
    nr = (cnt > 0.0) | reach
    newly = nr & jnp.logical_not(reach)
    dist = jnp.where(newly, float(k), dist)
    reach = nr
  spdsh_ref[...] = jnp.minimum(dist + 1.0, float(MAX_SPD + 1))


def _tc_spatial_body(spdsh_ref, wexp_ref, s128_ref, out_ref):
  f32 = jnp.float32
  d = spdsh_ref[...]
  s128 = s128_ref[...].astype(jnp.bfloat16)
  for cc in range(8):
    dexp = jnp.dot(d[:, cc * 128:(cc + 1) * 128].astype(jnp.bfloat16), s128,
                   preferred_element_type=f32)
    acc = jnp.zeros((128, N), f32)
    for k in range(1, MAX_SPD + 2):
      w_row = wexp_ref[k, :][None, :]
      acc = acc + jnp.where(dexp == float(k), w_row, 0.0)
    out_ref[:, pl.ds(cc * N, N)] = acc


def _sc_edge_body(src_hbm, dst_hbm, eb_hbm, neg1_hbm, spd_hbm, spdw_hbm, buf,
                  sbuf, dbuf, winner, sel_slot,
                  rowbuf, ebbuf, bidx, eidx, dist_v, spdw_v, sem):
  core = lax.axis_index("c")
  sid = lax.axis_index("s")
  wid = core * NSUB + sid
  i32 = jnp.int32

  pltpu.sync_copy(neg1_hbm, winner)
  pltpu.sync_copy(spd_hbm.at[pl.ds(wid * SLOTS_PER_W, SLOTS_PER_W)], dist_v)
  pltpu.sync_copy(spdw_hbm, spdw_v)

  iota16 = lax.iota(i32, 16)
  UNROLL = 8

  c2 = jnp.int32(0)
  for ch in range(NCHUNKS):
    pltpu.sync_copy(src_hbm.at[pl.ds(ch * CHUNK, CHUNK)], sbuf)
    pltpu.sync_copy(dst_hbm.at[pl.ds(ch * CHUNK, CHUNK)], dbuf)

    def scan(v, cnt, _ch=ch):
      ms, lslots, e16s, bads = [], [], [], []
      for q in range(UNROLL):
        s16 = sbuf[pl.ds((v * UNROLL + q) * 16, 16)]
        d16 = dbuf[pl.ds((v * UNROLL + q) * 16, 16)]
        m = jnp.right_shift(s16, 5) == wid
        lslot = jnp.bitwise_and(s16, 31) * N + d16
        e16 = (_ch * CHUNK + (v * UNROLL + q) * 16) + iota16

        plsc.store_scatter(winner, [lslot], e16, mask=m)
        wb = plsc.load_gather(winner, [lslot], mask=m)
        am = jnp.logical_and(m, wb == e16)
        plsc.store_compressed(sel_slot.at[pl.ds(cnt, 16)], lslot, mask=am)
        cnt = cnt + jnp.sum(am.astype(jnp.int32))
        ms.append(m); lslots.append(lslot); e16s.append(e16)
        bads.append(jnp.logical_and(m, wb < e16))
      badall = bads[0]
      for q in range(1, UNROLL):
        badall = jnp.logical_or(badall, bads[q])

      @pl.when(jnp.any(badall))
      def _():
        for q in range(UNROLL):
          def cond(b):
            return jnp.any(b > 0)

          def fix(b, _q=q):
            bm = b > 0
            plsc.store_scatter(winner, [lslots[_q]], e16s[_q], mask=bm)
            wb2 = plsc.load_gather(winner, [lslots[_q]], mask=bm)
            return jnp.logical_and(bm, wb2 < e16s[_q]).astype(i32)

          lax.while_loop(cond, fix, bads[q].astype(i32))
      return cnt
    c2 = lax.fori_loop(0, CHUNK // (16 * UNROLL), scan, c2)

  @pl.when(c2 > 0)
  def _():
    lastq = jnp.maximum(c2 - 1, 0)
    lasts = plsc.load_gather(sel_slot, [jnp.full((16,), lastq, i32)])
    sel_slot[pl.ds(c2, 16)] = lasts

  def apply16(j, _):
    lslot16 = sel_slot[pl.ds(j * 16, 16)]
    w16 = plsc.load_gather(winner, [lslot16])
    d16 = plsc.load_gather(dist_v, [lslot16]).astype(i32)
    base8 = (wid * SLOTS_PER_W + lslot16) * H
    ebase8 = w16 * H
    for hh in range(H):
      bidx[pl.ds(hh * 16, 16)] = base8 + hh
      eidx[pl.ds(hh * 16, 16)] = ebase8 + hh
      rowbuf[pl.ds(hh * 16, 16)] = plsc.load_gather(spdw_v, [d16 * H + hh])
    pltpu.async_copy(eb_hbm.at[eidx], ebbuf, sem).wait()
    for hh in range(H):
      v = rowbuf[pl.ds(hh * 16, 16)] + ebbuf[pl.ds(hh * 16, 16)]
      rowbuf[pl.ds(hh * 16, 16)] = v
    pltpu.async_copy(rowbuf, buf.at[bidx], sem).wait()
    return 0
  ngroups = (c2 + 15) // 16
  lax.fori_loop(0, ngroups, apply16, 0)


@functools.cache
def _sc_edge():
  return pl.kernel(
    _sc_edge_body,
    out_type=(),
    mesh=_mesh(),
    scratch_types=[
        pltpu.VMEM((CHUNK,), jnp.int32),
        pltpu.VMEM((CHUNK,), jnp.int32),
        pltpu.VMEM((SLOTS_PER_W,), jnp.int32),
        pltpu.VMEM((SLOTS_PER_W + 16,), jnp.int32),
        pltpu.VMEM((16 * H,), jnp.float32),
        pltpu.VMEM((16 * H,), jnp.float32),
        pltpu.VMEM((16 * H,), jnp.int32),
        pltpu.VMEM((16 * H,), jnp.int32),
        pltpu.VMEM((SLOTS_PER_W,), jnp.float32),
        pltpu.VMEM(((MAX_SPD + 2) * H,), jnp.float32),
        pltpu.SemaphoreType.DMA,
    ],
    compiler_params=pltpu.CompilerParams(needs_layout_passes=False),
  )


_div_term = np.exp(np.arange(0, D, 2, dtype=np.float64)
                   * (-np.log(10000.0) / D)).astype(np.float32)
_FREQ_COL = np.repeat(_div_term, 2).reshape(D, 1)
_SINM_COL = np.tile(np.array([1.0, 0.0], np.float32), D // 2).reshape(D, 1)
_S128 = np.zeros((128, N), np.float32)
for _c in range(128):
  _S128[_c, _c * 8:_c * 8 + 8] = 1.0


def kernel(edge_index, edge_attr, timestamps, deg_w, out_deg_w, spd_w,
           e_w1, e_b1, e_w2, e_b2, t_w, t_b, num_nodes):
  f32 = jnp.float32
  src = edge_index[0].astype(jnp.int32)
  dst = edge_index[1].astype(jnp.int32)
  n = timestamps.shape[0]

  adjp, indeg_p, outdeg_p = _sc_prep()(src, dst)

  extra = (jnp.asarray(num_nodes) - n).astype(f32).reshape(1, 1)
  dwT = jnp.zeros((D, D), f32).at[:, :MAX_DEG + 1].set(deg_w.T)
  odwT = jnp.zeros((D, D), f32).at[:, :MAX_DEG + 1].set(out_deg_w.T)
  ea_pad = jnp.pad(edge_attr, ((0, 0), (0, 5)))
  w1p = jnp.pad(e_w1, ((0, 5), (0, 0)))
  ts_row = timestamps.reshape(1, N)

  nodeT, eb = pl.pallas_call(
      _tc_node_body,
      out_shape=(
          jax.ShapeDtypeStruct((D, N), f32),
          jax.ShapeDtypeStruct((E, H), f32),
      ),
  )(indeg_p, outdeg_p, ts_row, extra,
    dwT, odwT, jnp.asarray(_FREQ_COL), jnp.asarray(_SINM_COL),
    t_w.T, t_b.reshape(D, 1),
    ea_pad, w1p, e_b1.reshape(1, 2 * H), e_w2, e_b2.reshape(1, H))

  spdsh = pl.pallas_call(
      _tc_bfs_body,
      out_shape=jax.ShapeDtypeStruct((N, N), f32),
  )(adjp.reshape(NCORES, N, N))

  wexp = jnp.tile(spd_w, (1, 128))
  attn2 = pl.pallas_call(
      _tc_spatial_body,
      grid=(8,),
      in_specs=[
          pl.BlockSpec((128, N), lambda i: (i, 0)),
          pl.BlockSpec((MAX_SPD + 2, N), lambda i: (0, 0)),
          pl.BlockSpec((128, N), lambda i: (0, 0)),
      ],
      out_specs=pl.BlockSpec((128, N * H), lambda i: (i, 0)),
      out_shape=jax.ShapeDtypeStruct((N, N * H), f32),
  )(spdsh, wexp, jnp.asarray(_S128))

  buf = jax.new_ref(attn2.reshape(N * N * H))
  neg1 = jnp.full((SLOTS_PER_W,), -1, jnp.int32)
  _sc_edge()(src, dst, eb.reshape(E * H), neg1,
             spdsh.reshape(N * N), spd_w.reshape((MAX_SPD + 2) * H), buf)
  attn_bias = buf[...].reshape(N, N, H)

  node_enc = nodeT.T
  return node_enc, attn_bias


# speedup vs baseline: 1.0088x; 1.0088x over previous
import functools

import jax
import jax.numpy as jnp
import numpy as np
from jax import lax
from jax.experimental import pallas as pl
from jax.experimental.pallas import tpu as pltpu
from jax.experimental.pallas import tpu_sc as plsc

N = 1024
E = 32768
H = 8
D = 128
MAX_DEG = 100
MAX_SPD = 10
MAX_TIME = 365.0

NCORES = 2
NSUB = 16
NW = NCORES * NSUB
ROWS_PER_W = N // NW
EPW = E // NW
SLOTS_PER_W = ROWS_PER_W * N
CHUNK = 4096
NCHUNKS = E // CHUNK

@functools.cache
def _mesh():
  return plsc.VectorSubcoreMesh(core_axis_name="c", subcore_axis_name="s",
                                num_cores=NCORES, num_subcores=NSUB)


def _sc_prep_body(src_hbm, dst_hbm, adj_out, indeg_out, outdeg_out,
                  src_v, dst_v, sidx2, didx2, idx1_2, idx2_2, ones_v, zbuf,
                  adj_sp, indeg_sp, outdeg_sp):
  core = lax.axis_index("c")
  sid = lax.axis_index("s")
  wid = core * NSUB + sid

  def fill(i, _):
    zbuf[pl.ds(i * 16, 16)] = jnp.zeros((16,), jnp.float32)
    return 0
  lax.fori_loop(0, 2048 // 16, fill, 0)

  def fill1(i, _):
    ones_v[pl.ds(i * 16, 16)] = jnp.ones((16,), jnp.float32)
    return 0
  lax.fori_loop(0, 128 // 16, fill1, 0)

  def z_adj(k, _):
    pltpu.sync_copy(zbuf, adj_sp.at[pl.ds(sid * (N * N // NSUB) + k * 2048, 2048)])
    return 0
  lax.fori_loop(0, (N * N // NSUB) // 2048, z_adj, 0)

  @pl.when(sid == 0)
  def _():
    pltpu.sync_copy(zbuf.at[pl.ds(0, N)], indeg_sp)
    pltpu.sync_copy(zbuf.at[pl.ds(0, N)], outdeg_sp)

  pltpu.sync_copy(src_hbm.at[pl.ds(wid * EPW, EPW)], src_v)
  pltpu.sync_copy(dst_hbm.at[pl.ds(wid * EPW, EPW)], dst_v)

  for j in range(EPW // 128):
    for q in range(8):
      i = j * 8 + q
      s16 = src_v[pl.ds(i * 16, 16)]
      d16 = dst_v[pl.ds(i * 16, 16)]
      sidx2[j, pl.ds(q * 16, 16)] = s16
      didx2[j, pl.ds(q * 16, 16)] = d16
      idx1_2[j, pl.ds(q * 16, 16)] = s16 * N + d16
      idx2_2[j, pl.ds(q * 16, 16)] = d16 * N + s16

  plsc.subcore_barrier()

  for j in range(EPW // 128):
    pltpu.sync_copy(ones_v, adj_sp.at[idx1_2.at[j]], add=True)
    pltpu.sync_copy(ones_v, adj_sp.at[idx2_2.at[j]], add=True)
    pltpu.sync_copy(ones_v, indeg_sp.at[didx2.at[j]], add=True)
    pltpu.sync_copy(ones_v, outdeg_sp.at[sidx2.at[j]], add=True)

  plsc.subcore_barrier()

  seg = N * N // NSUB
  pltpu.sync_copy(adj_sp.at[pl.ds(sid * seg, seg)],
                  adj_out.at[core, pl.ds(sid * seg, seg)])

  @pl.when(sid == 0)
  def _():
    pltpu.sync_copy(indeg_sp, indeg_out.at[core])
    pltpu.sync_copy(outdeg_sp, outdeg_out.at[core])


@functools.cache
def _sc_prep():
  return pl.kernel(
    _sc_prep_body,
    out_type=(
        jax.ShapeDtypeStruct((NCORES, N * N), jnp.float32),
        jax.ShapeDtypeStruct((NCORES, N), jnp.float32),
        jax.ShapeDtypeStruct((NCORES, N), jnp.float32),
    ),
    mesh=_mesh(),
    scratch_types=[
        pltpu.VMEM((EPW,), jnp.int32),
        pltpu.VMEM((EPW,), jnp.int32),
        pltpu.VMEM((EPW // 128, 128), jnp.int32),
        pltpu.VMEM((EPW // 128, 128), jnp.int32),
        pltpu.VMEM((EPW // 128, 128), jnp.int32),
        pltpu.VMEM((EPW // 128, 128), jnp.int32),
        pltpu.VMEM((128,), jnp.float32),
        pltpu.VMEM((2048,), jnp.float32),
        pltpu.VMEM_SHARED((N * N,), jnp.float32),
        pltpu.VMEM_SHARED((N,), jnp.float32),
        pltpu.VMEM_SHARED((N,), jnp.float32),
    ],
    compiler_params=pltpu.CompilerParams(needs_layout_passes=False),
  )


def _tc_node_body(indeg_ref, outdeg_ref, ts_ref, extra_ref,
                  dwT_ref, odwT_ref, freq_ref, sinm_ref, twT_ref, tb_ref,
                  ea_ref, w1_ref, b1_ref, w2_ref, b2_ref,
                  nodeT_ref, eb_ref):
  f32 = jnp.float32
  extra = extra_ref[0, 0]

  ind = indeg_ref[...]
  ind_row = ind[0:1, :] + ind[1:2, :] + extra
  ind_row = jnp.minimum(ind_row, float(MAX_DEG))
  outd = outdeg_ref[...]
  outd_row = jnp.minimum(outd[0:1, :] + outd[1:2, :], float(MAX_DEG))

  iota_col = lax.broadcasted_iota(jnp.int32, (D, N), 0)
  ohT_in = (iota_col == ind_row.astype(jnp.int32)).astype(f32)
  ohT_out = (iota_col == outd_row.astype(jnp.int32)).astype(f32)
  centT = (jnp.dot(dwT_ref[...], ohT_in, preferred_element_type=f32)
           + jnp.dot(odwT_ref[...], ohT_out, preferred_element_type=f32))

  ts = ts_ref[...]
  tmin = jnp.min(ts)
  td = jnp.minimum((ts - tmin) / 86400.0, MAX_TIME)
  prod = freq_ref[...] * td
  peT = jnp.where(sinm_ref[...] > 0.5, jnp.sin(prod), jnp.cos(prod))
  tempT = jnp.dot(twT_ref[...], peT, preferred_element_type=f32) + tb_ref[...]

  nodeT_ref[...] = centT + tempT

  h = jnp.maximum(
      jnp.dot(ea_ref[...], w1_ref[...], preferred_element_type=f32)
      + b1_ref[...], 0.0)
  eb_ref[...] = jnp.dot(h, w2_ref[...], preferred_element_type=f32) + b2_ref[...]


def _tc_bfs_body(adjp_ref, spdsh_ref):
  f32 = jnp.float32
  a = adjp_ref[...]
  adj_b = (a[0] + a[1]) > 0.0
  r = lax.broadcasted_iota(jnp.int32, (N, N), 0)
  c = lax.broadcasted_iota(jnp.int32, (N, N), 1)
  eye = r == c
  reach = adj_b | eye
  dist = jnp.where(eye, 0.0, jnp.where(adj_b, 1.0, float(MAX_SPD + 1)))
  adj_bf = adj_b.astype(jnp.bfloat16)
  for k in range(2, MAX_SPD + 1):
    cnt = jnp.dot(reach.astype(jnp.bfloat16), adj_bf,
                  preferred_element_type=f32)
    nr = (cnt > 0.0) | reach
    newly = nr & jnp.logical_not(reach)
    dist = jnp.where(newly, float(k), dist)
    reach = nr
  spdsh_ref[...] = jnp.minimum(dist + 1.0, float(MAX_SPD + 1))


def _tc_spatial_body(spdsh_ref, wexp_ref, s128_ref, out_ref):
  f32 = jnp.float32
  d = spdsh_ref[...]
  s128 = s128_ref[...].astype(jnp.bfloat16)
  for cc in range(8):
    dexp = jnp.dot(d[:, cc * 128:(cc + 1) * 128].astype(jnp.bfloat16), s128,
                   preferred_element_type=f32)
    acc = jnp.zeros((128, N), f32)
    for k in range(1, MAX_SPD + 2):
      w_row = wexp_ref[k, :][None, :]
      acc = acc + jnp.where(dexp == float(k), w_row, 0.0)
    out_ref[:, pl.ds(cc * N, N)] = acc


def _sc_edge_body(src_hbm, dst_hbm, eb_hbm, neg1_hbm, buf,
                  sbuf, dbuf, winner, sel_slot,
                  rowbuf, ebbuf, bidx, eidx, sem):
  core = lax.axis_index("c")
  sid = lax.axis_index("s")
  wid = core * NSUB + sid
  i32 = jnp.int32

  pltpu.sync_copy(neg1_hbm, winner)

  iota16 = lax.iota(i32, 16)
  UNROLL = 8

  c2 = jnp.int32(0)
  for ch in range(NCHUNKS):
    pltpu.sync_copy(src_hbm.at[pl.ds(ch * CHUNK, CHUNK)], sbuf)
    pltpu.sync_copy(dst_hbm.at[pl.ds(ch * CHUNK, CHUNK)], dbuf)

    def scan(v, cnt, _ch=ch):
      ms, lslots, e16s, bads = [], [], [], []
      for q in range(UNROLL):
        s16 = sbuf[pl.ds((v * UNROLL + q) * 16, 16)]
        d16 = dbuf[pl.ds((v * UNROLL + q) * 16, 16)]
        m = jnp.right_shift(s16, 5) == wid
        lslot = jnp.bitwise_and(s16, 31) * N + d16
        e16 = (_ch * CHUNK + (v * UNROLL + q) * 16) + iota16

        pre = plsc.load_gather(winner, [lslot], mask=m)
        plsc.store_scatter(winner, [lslot], e16, mask=m)
        wb = plsc.load_gather(winner, [lslot], mask=m)
        am = jnp.logical_and(jnp.logical_and(m, pre == -1), wb == e16)
        plsc.store_compressed(sel_slot.at[pl.ds(cnt, 16)], lslot, mask=am)
        cnt = cnt + jnp.sum(am.astype(jnp.int32))
        ms.append(m); lslots.append(lslot); e16s.append(e16)
        bads.append(jnp.logical_and(m, wb < e16))
      badall = bads[0]
      for q in range(1, UNROLL):
        badall = jnp.logical_or(badall, bads[q])

      @pl.when(jnp.any(badall))
      def _():
        for q in range(UNROLL):
          def cond(b):
            return jnp.any(b > 0)

          def fix(b, _q=q):
            bm = b > 0
            plsc.store_scatter(winner, [lslots[_q]], e16s[_q], mask=bm)
            wb2 = plsc.load_gather(winner, [lslots[_q]], mask=bm)
            return jnp.logical_and(bm, wb2 < e16s[_q]).astype(i32)

          lax.while_loop(cond, fix, bads[q].astype(i32))
      return cnt
    c2 = lax.fori_loop(0, CHUNK // (16 * UNROLL), scan, c2)

  @pl.when(c2 > 0)
  def _():
    lastq = jnp.maximum(c2 - 1, 0)
    lasts = plsc.load_gather(sel_slot, [jnp.full((16,), lastq, i32)])
    sel_slot[pl.ds(c2, 16)] = lasts

  def apply16(j, _):
    lslot16 = sel_slot[pl.ds(j * 16, 16)]
    w16 = plsc.load_gather(winner, [lslot16])
    base8 = (wid * SLOTS_PER_W + lslot16) * H
    ebase8 = w16 * H
    for hh in range(H):
      bidx[pl.ds(hh * 16, 16)] = base8 + hh
      eidx[pl.ds(hh * 16, 16)] = ebase8 + hh
    d1 = pltpu.async_copy(buf.at[bidx], rowbuf, sem)
    d2 = pltpu.async_copy(eb_hbm.at[eidx], ebbuf, sem)
    d1.wait()
    d2.wait()
    for hh in range(H):
      v = rowbuf[pl.ds(hh * 16, 16)] + ebbuf[pl.ds(hh * 16, 16)]
      rowbuf[pl.ds(hh * 16, 16)] = v
    pltpu.async_copy(rowbuf, buf.at[bidx], sem).wait()
    return 0
  ngroups = (c2 + 15) // 16
  lax.fori_loop(0, ngroups, apply16, 0)


@functools.cache
def _sc_edge():
  return pl.kernel(
    _sc_edge_body,
    out_type=(),
    mesh=_mesh(),
    scratch_types=[
        pltpu.VMEM((CHUNK,), jnp.int32),
        pltpu.VMEM((CHUNK,), jnp.int32),
        pltpu.VMEM((SLOTS_PER_W,), jnp.int32),
        pltpu.VMEM((SLOTS_PER_W + 16,), jnp.int32),
        pltpu.VMEM((16 * H,), jnp.float32),
        pltpu.VMEM((16 * H,), jnp.float32),
        pltpu.VMEM((16 * H,), jnp.int32),
        pltpu.VMEM((16 * H,), jnp.int32),
        pltpu.SemaphoreType.DMA,
    ],
    compiler_params=pltpu.CompilerParams(needs_layout_passes=False),
  )


_div_term = np.exp(np.arange(0, D, 2, dtype=np.float64)
                   * (-np.log(10000.0) / D)).astype(np.float32)
_FREQ_COL = np.repeat(_div_term, 2).reshape(D, 1)
_SINM_COL = np.tile(np.array([1.0, 0.0], np.float32), D // 2).reshape(D, 1)
_S128 = np.zeros((128, N), np.float32)
for _c in range(128):
  _S128[_c, _c * 8:_c * 8 + 8] = 1.0


def kernel(edge_index, edge_attr, timestamps, deg_w, out_deg_w, spd_w,
           e_w1, e_b1, e_w2, e_b2, t_w, t_b, num_nodes):
  f32 = jnp.float32
  src = edge_index[0].astype(jnp.int32)
  dst = edge_index[1].astype(jnp.int32)
  n = timestamps.shape[0]

  adjp, indeg_p, outdeg_p = _sc_prep()(src, dst)

  extra = (jnp.asarray(num_nodes) - n).astype(f32).reshape(1, 1)
  dwT = jnp.zeros((D, D), f32).at[:, :MAX_DEG + 1].set(deg_w.T)
  odwT = jnp.zeros((D, D), f32).at[:, :MAX_DEG + 1].set(out_deg_w.T)
  ea_pad = jnp.pad(edge_attr, ((0, 0), (0, 5)))
  w1p = jnp.pad(e_w1, ((0, 5), (0, 0)))
  ts_row = timestamps.reshape(1, N)

  nodeT, eb = pl.pallas_call(
      _tc_node_body,
      out_shape=(
          jax.ShapeDtypeStruct((D, N), f32),
          jax.ShapeDtypeStruct((E, H), f32),
      ),
  )(indeg_p, outdeg_p, ts_row, extra,
    dwT, odwT, jnp.asarray(_FREQ_COL), jnp.asarray(_SINM_COL),
    t_w.T, t_b.reshape(D, 1),
    ea_pad, w1p, e_b1.reshape(1, 2 * H), e_w2, e_b2.reshape(1, H))

  spdsh = pl.pallas_call(
      _tc_bfs_body,
      out_shape=jax.ShapeDtypeStruct((N, N), f32),
  )(adjp.reshape(NCORES, N, N))

  wexp = jnp.tile(spd_w, (1, 128))
  attn2 = pl.pallas_call(
      _tc_spatial_body,
      grid=(8,),
      in_specs=[
          pl.BlockSpec((128, N), lambda i: (i, 0)),
          pl.BlockSpec((MAX_SPD + 2, N), lambda i: (0, 0)),
          pl.BlockSpec((128, N), lambda i: (0, 0)),
      ],
      out_specs=pl.BlockSpec((128, N * H), lambda i: (i, 0)),
      out_shape=jax.ShapeDtypeStruct((N, N * H), f32),
  )(spdsh, wexp, jnp.asarray(_S128))

  buf = jax.new_ref(attn2.reshape(N * N * H))
  neg1 = jnp.full((SLOTS_PER_W,), -1, jnp.int32)
  _sc_edge()(src, dst, eb.reshape(E * H), neg1, buf)
  attn_bias = buf[...].reshape(N, N, H)

  node_enc = nodeT.T
  return node_enc, attn_bias


# speedup vs baseline: 1.0115x; 1.0027x over previous
import functools

import jax
import jax.numpy as jnp
import numpy as np
from jax import lax
from jax.experimental import pallas as pl
from jax.experimental.pallas import tpu as pltpu
from jax.experimental.pallas import tpu_sc as plsc

N = 1024
E = 32768
H = 8
D = 128
MAX_DEG = 100
MAX_SPD = 10
MAX_TIME = 365.0

NCORES = 2
NSUB = 16
NW = NCORES * NSUB
ROWS_PER_W = N // NW
EPW = E // NW
SLOTS_PER_W = ROWS_PER_W * N
CHUNK = 4096
NCHUNKS = E // CHUNK

@functools.cache
def _mesh():
  return plsc.VectorSubcoreMesh(core_axis_name="c", subcore_axis_name="s",
                                num_cores=NCORES, num_subcores=NSUB)


def _sc_prep_body(src_hbm, dst_hbm, adj_out, indeg_out, outdeg_out,
                  src_v, dst_v, sidx2, didx2, idx1_2, idx2_2, ones_v, zbuf,
                  adj_sp, indeg_sp, outdeg_sp):
  core = lax.axis_index("c")
  sid = lax.axis_index("s")
  wid = core * NSUB + sid

  def fill(i, _):
    zbuf[pl.ds(i * 16, 16)] = jnp.zeros((16,), jnp.float32)
    return 0
  lax.fori_loop(0, 2048 // 16, fill, 0)

  def fill1(i, _):
    ones_v[pl.ds(i * 16, 16)] = jnp.ones((16,), jnp.float32)
    return 0
  lax.fori_loop(0, 128 // 16, fill1, 0)

  def z_adj(k, _):
    pltpu.sync_copy(zbuf, adj_sp.at[pl.ds(sid * (N * N // NSUB) + k * 2048, 2048)])
    return 0
  lax.fori_loop(0, (N * N // NSUB) // 2048, z_adj, 0)

  @pl.when(sid == 0)
  def _():
    pltpu.sync_copy(zbuf.at[pl.ds(0, N)], indeg_sp)
    pltpu.sync_copy(zbuf.at[pl.ds(0, N)], outdeg_sp)

  pltpu.sync_copy(src_hbm.at[pl.ds(wid * EPW, EPW)], src_v)
  pltpu.sync_copy(dst_hbm.at[pl.ds(wid * EPW, EPW)], dst_v)

  for j in range(EPW // 128):
    for q in range(8):
      i = j * 8 + q
      s16 = src_v[pl.ds(i * 16, 16)]
      d16 = dst_v[pl.ds(i * 16, 16)]
      sidx2[j, pl.ds(q * 16, 16)] = s16
      didx2[j, pl.ds(q * 16, 16)] = d16
      idx1_2[j, pl.ds(q * 16, 16)] = s16 * N + d16
      idx2_2[j, pl.ds(q * 16, 16)] = d16 * N + s16

  plsc.subcore_barrier()

  for j in range(EPW // 128):
    pltpu.sync_copy(ones_v, adj_sp.at[idx1_2.at[j]], add=True)
    pltpu.sync_copy(ones_v, adj_sp.at[idx2_2.at[j]], add=True)
    pltpu.sync_copy(ones_v, indeg_sp.at[didx2.at[j]], add=True)
    pltpu.sync_copy(ones_v, outdeg_sp.at[sidx2.at[j]], add=True)

  plsc.subcore_barrier()

  seg = N * N // NSUB
  pltpu.sync_copy(adj_sp.at[pl.ds(sid * seg, seg)],
                  adj_out.at[core, pl.ds(sid * seg, seg)])

  @pl.when(sid == 0)
  def _():
    pltpu.sync_copy(indeg_sp, indeg_out.at[core])
    pltpu.sync_copy(outdeg_sp, outdeg_out.at[core])


@functools.cache
def _sc_prep():
  return pl.kernel(
    _sc_prep_body,
    out_type=(
        jax.ShapeDtypeStruct((NCORES, N * N), jnp.float32),
        jax.ShapeDtypeStruct((NCORES, N), jnp.float32),
        jax.ShapeDtypeStruct((NCORES, N), jnp.float32),
    ),
    mesh=_mesh(),
    scratch_types=[
        pltpu.VMEM((EPW,), jnp.int32),
        pltpu.VMEM((EPW,), jnp.int32),
        pltpu.VMEM((EPW // 128, 128), jnp.int32),
        pltpu.VMEM((EPW // 128, 128), jnp.int32),
        pltpu.VMEM((EPW // 128, 128), jnp.int32),
        pltpu.VMEM((EPW // 128, 128), jnp.int32),
        pltpu.VMEM((128,), jnp.float32),
        pltpu.VMEM((2048,), jnp.float32),
        pltpu.VMEM_SHARED((N * N,), jnp.float32),
        pltpu.VMEM_SHARED((N,), jnp.float32),
        pltpu.VMEM_SHARED((N,), jnp.float32),
    ],
    compiler_params=pltpu.CompilerParams(needs_layout_passes=False),
  )


def _tc_node_body(indeg_ref, outdeg_ref, ts_ref, extra_ref,
                  dwT_ref, odwT_ref, freq_ref, sinm_ref, twT_ref, tb_ref,
                  ea_ref, w1_ref, b1_ref, w2_ref, b2_ref,
                  nodeT_ref, eb_ref):
  f32 = jnp.float32
  extra = extra_ref[0, 0]

  ind = indeg_ref[...]
  ind_row = ind[0:1, :] + ind[1:2, :] + extra
  ind_row = jnp.minimum(ind_row, float(MAX_DEG))
  outd = outdeg_ref[...]
  outd_row = jnp.minimum(outd[0:1, :] + outd[1:2, :], float(MAX_DEG))

  iota_col = lax.broadcasted_iota(jnp.int32, (D, N), 0)
  ohT_in = (iota_col == ind_row.astype(jnp.int32)).astype(f32)
  ohT_out = (iota_col == outd_row.astype(jnp.int32)).astype(f32)
  centT = (jnp.dot(dwT_ref[...], ohT_in, preferred_element_type=f32)
           + jnp.dot(odwT_ref[...], ohT_out, preferred_element_type=f32))

  ts = ts_ref[...]
  tmin = jnp.min(ts)
  td = jnp.minimum((ts - tmin) / 86400.0, MAX_TIME)
  prod = freq_ref[...] * td
  peT = jnp.where(sinm_ref[...] > 0.5, jnp.sin(prod), jnp.cos(prod))
  tempT = jnp.dot(twT_ref[...], peT, preferred_element_type=f32) + tb_ref[...]

  nodeT_ref[...] = centT + tempT

  h = jnp.maximum(
      jnp.dot(ea_ref[...], w1_ref[...], preferred_element_type=f32)
      + b1_ref[...], 0.0)
  eb_ref[...] = jnp.dot(h, w2_ref[...], preferred_element_type=f32) + b2_ref[...]


def _tc_bfs_body(adjp_ref, spdsh_ref):
  f32 = jnp.float32
  a = adjp_ref[...]
  adj_b = (a[0] + a[1]) > 0.0
  r = lax.broadcasted_iota(jnp.int32, (N, N), 0)
  c = lax.broadcasted_iota(jnp.int32, (N, N), 1)
  eye = r == c
  reach = adj_b | eye
  dist = jnp.where(eye, 0.0, jnp.where(adj_b, 1.0, float(MAX_SPD + 1)))
  adj_bf = adj_b.astype(jnp.bfloat16)
  for k in range(2, MAX_SPD + 1):
    cnt = jnp.dot(reach.astype(jnp.bfloat16), adj_bf,
                  preferred_element_type=f32)
    nr = (cnt > 0.0) | reach
    newly = nr & jnp.logical_not(reach)
    dist = jnp.where(newly, float(k), dist)
    reach = nr
  spdsh_ref[...] = jnp.minimum(dist + 1.0, float(MAX_SPD + 1))


def _tc_spatial_body(spdsh_ref, wexp_ref, s128_ref, out_ref):
  f32 = jnp.float32
  d = spdsh_ref[...]
  s128 = s128_ref[...].astype(jnp.bfloat16)
  for cc in range(8):
    dexp = jnp.dot(d[:, cc * 128:(cc + 1) * 128].astype(jnp.bfloat16), s128,
                   preferred_element_type=f32)
    acc = jnp.zeros((128, N), f32)
    for k in range(1, MAX_SPD + 2):
      w_row = wexp_ref[k, :][None, :]
      acc = acc + jnp.where(dexp == float(k), w_row, 0.0)
    out_ref[:, pl.ds(cc * N, N)] = acc


def _sc_edge_body(src_hbm, dst_hbm, eb_hbm, neg1_hbm, buf,
                  sbuf, dbuf, winner, sel_slot,
                  rowbuf, ebbuf, bidx, eidx, sem):
  core = lax.axis_index("c")
  sid = lax.axis_index("s")
  wid = core * NSUB + sid
  i32 = jnp.int32

  pltpu.sync_copy(neg1_hbm, winner)

  iota16 = lax.iota(i32, 16)
  UNROLL = 8

  c2 = jnp.int32(0)
  for ch in range(NCHUNKS):
    pltpu.sync_copy(src_hbm.at[pl.ds(ch * CHUNK, CHUNK)], sbuf)
    pltpu.sync_copy(dst_hbm.at[pl.ds(ch * CHUNK, CHUNK)], dbuf)

    def scan(v, cnt, _ch=ch):
      ms, lslots, e16s, bads = [], [], [], []
      for q in range(UNROLL):
        s16 = sbuf[pl.ds((v * UNROLL + q) * 16, 16)]
        d16 = dbuf[pl.ds((v * UNROLL + q) * 16, 16)]
        m = jnp.right_shift(s16, 5) == wid
        lslot = jnp.bitwise_and(s16, 31) * N + d16
        e16 = (_ch * CHUNK + (v * UNROLL + q) * 16) + iota16

        pre = plsc.load_gather(winner, [lslot], mask=m)
        plsc.store_scatter(winner, [lslot], e16, mask=m)
        wb = plsc.load_gather(winner, [lslot], mask=m)
        am = jnp.logical_and(jnp.logical_and(m, pre == -1), wb == e16)
        plsc.store_compressed(sel_slot.at[pl.ds(cnt, 16)], lslot, mask=am)
        cnt = cnt + plsc.all_reduce_population_count(am)[0]
        ms.append(m); lslots.append(lslot); e16s.append(e16)
        bads.append(jnp.logical_and(m, wb < e16))
      badall = bads[0]
      for q in range(1, UNROLL):
        badall = jnp.logical_or(badall, bads[q])

      @pl.when(jnp.any(badall))
      def _():
        for q in range(UNROLL):
          def cond(b):
            return jnp.any(b > 0)

          def fix(b, _q=q):
            bm = b > 0
            plsc.store_scatter(winner, [lslots[_q]], e16s[_q], mask=bm)
            wb2 = plsc.load_gather(winner, [lslots[_q]], mask=bm)
            return jnp.logical_and(bm, wb2 < e16s[_q]).astype(i32)

          lax.while_loop(cond, fix, bads[q].astype(i32))
      return cnt
    c2 = lax.fori_loop(0, CHUNK // (16 * UNROLL), scan, c2)

  @pl.when(c2 > 0)
  def _():
    lastq = jnp.maximum(c2 - 1, 0)
    lasts = plsc.load_gather(sel_slot, [jnp.full((16,), lastq, i32)])
    sel_slot[pl.ds(c2, 16)] = lasts

  def apply16(j, _):
    lslot16 = sel_slot[pl.ds(j * 16, 16)]
    w16 = plsc.load_gather(winner, [lslot16])
    base8 = (wid * SLOTS_PER_W + lslot16) * H
    ebase8 = w16 * H
    for hh in range(H):
      bidx[pl.ds(hh * 16, 16)] = base8 + hh
      eidx[pl.ds(hh * 16, 16)] = ebase8 + hh
    d1 = pltpu.async_copy(buf.at[bidx], rowbuf, sem)
    d2 = pltpu.async_copy(eb_hbm.at[eidx], ebbuf, sem)
    d1.wait()
    d2.wait()
    for hh in range(H):
      v = rowbuf[pl.ds(hh * 16, 16)] + ebbuf[pl.ds(hh * 16, 16)]
      rowbuf[pl.ds(hh * 16, 16)] = v
    pltpu.async_copy(rowbuf, buf.at[bidx], sem).wait()
    return 0
  ngroups = (c2 + 15) // 16
  lax.fori_loop(0, ngroups, apply16, 0)


@functools.cache
def _sc_edge():
  return pl.kernel(
    _sc_edge_body,
    out_type=(),
    mesh=_mesh(),
    scratch_types=[
        pltpu.VMEM((CHUNK,), jnp.int32),
        pltpu.VMEM((CHUNK,), jnp.int32),
        pltpu.VMEM((SLOTS_PER_W,), jnp.int32),
        pltpu.VMEM((SLOTS_PER_W + 16,), jnp.int32),
        pltpu.VMEM((16 * H,), jnp.float32),
        pltpu.VMEM((16 * H,), jnp.float32),
        pltpu.VMEM((16 * H,), jnp.int32),
        pltpu.VMEM((16 * H,), jnp.int32),
        pltpu.SemaphoreType.DMA,
    ],
    compiler_params=pltpu.CompilerParams(needs_layout_passes=False),
  )


_div_term = np.exp(np.arange(0, D, 2, dtype=np.float64)
                   * (-np.log(10000.0) / D)).astype(np.float32)
_FREQ_COL = np.repeat(_div_term, 2).reshape(D, 1)
_SINM_COL = np.tile(np.array([1.0, 0.0], np.float32), D // 2).reshape(D, 1)
_S128 = np.zeros((128, N), np.float32)
for _c in range(128):
  _S128[_c, _c * 8:_c * 8 + 8] = 1.0


def kernel(edge_index, edge_attr, timestamps, deg_w, out_deg_w, spd_w,
           e_w1, e_b1, e_w2, e_b2, t_w, t_b, num_nodes):
  f32 = jnp.float32
  src = edge_index[0].astype(jnp.int32)
  dst = edge_index[1].astype(jnp.int32)
  n = timestamps.shape[0]

  adjp, indeg_p, outdeg_p = _sc_prep()(src, dst)

  extra = (jnp.asarray(num_nodes) - n).astype(f32).reshape(1, 1)
  dwT = jnp.zeros((D, D), f32).at[:, :MAX_DEG + 1].set(deg_w.T)
  odwT = jnp.zeros((D, D), f32).at[:, :MAX_DEG + 1].set(out_deg_w.T)
  ea_pad = jnp.pad(edge_attr, ((0, 0), (0, 5)))
  w1p = jnp.pad(e_w1, ((0, 5), (0, 0)))
  ts_row = timestamps.reshape(1, N)

  nodeT, eb = pl.pallas_call(
      _tc_node_body,
      out_shape=(
          jax.ShapeDtypeStruct((D, N), f32),
          jax.ShapeDtypeStruct((E, H), f32),
      ),
  )(indeg_p, outdeg_p, ts_row, extra,
    dwT, odwT, jnp.asarray(_FREQ_COL), jnp.asarray(_SINM_COL),
    t_w.T, t_b.reshape(D, 1),
    ea_pad, w1p, e_b1.reshape(1, 2 * H), e_w2, e_b2.reshape(1, H))

  spdsh = pl.pallas_call(
      _tc_bfs_body,
      out_shape=jax.ShapeDtypeStruct((N, N), f32),
  )(adjp.reshape(NCORES, N, N))

  wexp = jnp.tile(spd_w, (1, 128))
  attn2 = pl.pallas_call(
      _tc_spatial_body,
      grid=(8,),
      in_specs=[
          pl.BlockSpec((128, N), lambda i: (i, 0)),
          pl.BlockSpec((MAX_SPD + 2, N), lambda i: (0, 0)),
          pl.BlockSpec((128, N), lambda i: (0, 0)),
      ],
      out_specs=pl.BlockSpec((128, N * H), lambda i: (i, 0)),
      out_shape=jax.ShapeDtypeStruct((N, N * H), f32),
  )(spdsh, wexp, jnp.asarray(_S128))

  buf = jax.new_ref(attn2.reshape(N * N * H))
  neg1 = jnp.full((SLOTS_PER_W,), -1, jnp.int32)
  _sc_edge()(src, dst, eb.reshape(E * H), neg1, buf)
  attn_bias = buf[...].reshape(N, N, H)

  node_enc = nodeT.T
  return node_enc, attn_bias
